# baseline (speedup 1.0000x reference)
import functools

import jax
import jax.numpy as jnp
from jax import lax
from jax.experimental import pallas as pl
from jax.experimental.pallas import tpu as pltpu
from jax.experimental.pallas import tpu_sc as plsc

NC = 2
NS = 16
LANE = 128
NCH = 8


def _chunks(ept):
    lb = (ept // NCH) // 16 * 16
    lens = [lb] * (NCH - 1) + [ept - (NCH - 1) * lb]
    offs = [i * lb for i in range(NCH)]
    return lens, offs


def _sc_degree(edge, w, zeros_np, np_, ept, eoff):
    slc = np_ // NS
    lens, offs = _chunks(ept)
    mesh = plsc.VectorSubcoreMesh(core_axis_name="c", subcore_axis_name="s",
                                  num_cores=NC, num_subcores=NS)

    @functools.partial(
        pl.kernel,
        out_type=jax.ShapeDtypeStruct((NC, np_), jnp.float32),
        mesh=mesh,
        scratch_types=[
            [pltpu.VMEM((l,), jnp.int32) for l in lens],
            pltpu.VMEM((ept,), jnp.float32),
            pltpu.SemaphoreType.DMA,
            pltpu.SemaphoreType.DMA,
            pltpu.VMEM_SHARED((np_,), jnp.float32),
        ],
    )
    def k(edge_h, w_h, z_h, out_h, col_c, w_v, stsem, ssem, acc):
        cid = lax.axis_index("c")
        sid = lax.axis_index("s")
        base = (sid * NC + cid) * ept
        ds = [pltpu.async_copy(edge_h.at[pl.ds(eoff + base + offs[i], lens[i])],
                               col_c[i], stsem) for i in range(NCH)]
        dw = pltpu.async_copy(w_h.at[pl.ds(base, ept)], w_v, stsem)
        pltpu.sync_copy(z_h.at[pl.ds(sid * slc, slc)], acc.at[pl.ds(sid * slc, slc)])
        for d in ds:
            d.wait()
        dw.wait()
        plsc.subcore_barrier()
        for i in range(NCH):
            pltpu.async_copy(w_v.at[pl.ds(offs[i], lens[i])], acc.at[col_c[i]],
                             ssem, add=True)
        for i in range(NCH):
            pltpu.make_async_copy(w_v.at[pl.ds(offs[i], lens[i])],
                                  acc.at[col_c[i]], ssem).wait()
        plsc.subcore_barrier()
        pltpu.sync_copy(acc.at[pl.ds(sid * slc, slc)],
                        out_h.at[cid, pl.ds(sid * slc, slc)])

    return k(edge, w, zeros_np)


def _sc_weighted_agg(edge, w, degp, xp, zeros_np, np_, ept, eoff):
    slc = np_ // NS
    lens, offs = _chunks(ept)
    lmax = -(-max(lens) // 16) * 16
    mesh = plsc.VectorSubcoreMesh(core_axis_name="c", subcore_axis_name="s",
                                  num_cores=NC, num_subcores=NS)

    @functools.partial(
        pl.kernel,
        out_type=(jax.ShapeDtypeStruct((NC, np_), jnp.float32),
                  jax.ShapeDtypeStruct((np_,), jnp.float32)),
        mesh=mesh,
        scratch_types=[
            [pltpu.VMEM((l,), jnp.int32) for l in lens],
            pltpu.VMEM((ept,), jnp.int32),
            pltpu.VMEM((ept,), jnp.float32),
            pltpu.VMEM((lmax,), jnp.float32),
            pltpu.VMEM((lmax,), jnp.float32),
            pltpu.VMEM((lmax,), jnp.float32),
            pltpu.VMEM((lmax,), jnp.float32),
            pltpu.VMEM((slc,), jnp.float32),
            pltpu.VMEM((slc,), jnp.float32),
            pltpu.VMEM((slc,), jnp.float32),
            pltpu.VMEM((slc,), jnp.float32),
            pltpu.VMEM((slc,), jnp.float32),
            pltpu.SemaphoreType.DMA,
            pltpu.SemaphoreType.DMA,
            pltpu.SemaphoreType.DMA,
            pltpu.SemaphoreType.DMA,
            pltpu.SemaphoreType.DMA,
            pltpu.VMEM_SHARED((np_,), jnp.float32),
            pltpu.VMEM_SHARED((np_,), jnp.float32),
        ],
    )
    def k(edge_h, w_h, degp_h, x_h, z_h, sp_h, dinv_h,
          col_c, row_v, w_v, uvA, uvB, tvA, tvB, p0v, p1v, xcv, ucv, dcv,
          stsem, gA, gB, sA, sB, u_sh, acc):
        cid = lax.axis_index("c")
        sid = lax.axis_index("s")
        base = (sid * NC + cid) * ept
        off = sid * slc
        dstage = [pltpu.async_copy(edge_h.at[pl.ds(eoff + base + offs[i], lens[i])],
                                   col_c[i], stsem) for i in range(NCH)]
        dstage.append(pltpu.async_copy(edge_h.at[pl.ds(base, ept)], row_v,
                                       stsem))
        dstage.append(pltpu.async_copy(w_h.at[pl.ds(base, ept)], w_v, stsem))
        pltpu.sync_copy(z_h.at[pl.ds(off, slc)], acc.at[pl.ds(off, slc)])
        pltpu.sync_copy(degp_h.at[0, pl.ds(off, slc)], p0v)
        pltpu.sync_copy(degp_h.at[1, pl.ds(off, slc)], p1v)
        pltpu.sync_copy(x_h.at[pl.ds(off, slc)], xcv)

        def newton(i, c):
            sl = pl.ds(i * 16, 16)
            d = p0v[sl] + p1v[sl] + 1.0
            iv = lax.bitcast_convert_type(d, jnp.int32)
            y = lax.bitcast_convert_type(
                jnp.full((16,), 0x5F3759DF, jnp.int32) - jnp.right_shift(iv, 1),
                jnp.float32)
            for _ in range(3):
                y = y * (1.5 - 0.5 * d * y * y)
            dcv[sl] = y
            ucv[sl] = y * xcv[sl]
            return c

        lax.fori_loop(0, slc // 16, newton, 0)
        pltpu.sync_copy(ucv, u_sh.at[pl.ds(off, slc)])

        @pl.when(cid == 0)
        def _():
            pltpu.sync_copy(dcv, dinv_h.at[pl.ds(off, slc)])

        plsc.subcore_barrier()
        for d in dstage:
            d.wait()

        def g_fire(ci, uvX, sem):
            return pltpu.async_copy(
                u_sh.at[row_v.at[pl.ds(offs[ci], lens[ci])]],
                uvX.at[pl.ds(0, lens[ci])], sem)

        def s_fire(ci, tvX, sem):
            return pltpu.async_copy(tvX.at[pl.ds(0, lens[ci])],
                                    acc.at[col_c[ci]], sem, add=True)

        def s_drain(ci, tvX, sem):
            pltpu.make_async_copy(tvX.at[pl.ds(0, lens[ci])],
                                  acc.at[col_c[ci]], sem).wait()

        def mul(ci, uvX, tvX):
            n16 = -(-lens[ci] // 16)

            def b(i, c):
                sl = pl.ds(i * 16, 16)
                tvX[sl] = w_v[pl.ds(offs[ci] + i * 16, 16)] * uvX[sl]
                return c

            lax.fori_loop(0, n16, b, 0)

        ga = g_fire(0, uvA, gA)
        gb = g_fire(1, uvB, gB)
        for c in range(0, NCH, 2):
            ga.wait()
            if c >= 2:
                s_drain(c - 2, tvA, sA)
            mul(c, uvA, tvA)
            s_fire(c, tvA, sA)
            if c + 2 < NCH:
                ga = g_fire(c + 2, uvA, gA)
            gb.wait()
            if c >= 2:
                s_drain(c - 1, tvB, sB)
            mul(c + 1, uvB, tvB)
            s_fire(c + 1, tvB, sB)
            if c + 3 < NCH:
                gb = g_fire(c + 3, uvB, gB)
        s_drain(NCH - 2, tvA, sA)
        s_drain(NCH - 1, tvB, sB)
        plsc.subcore_barrier()
        pltpu.sync_copy(acc.at[pl.ds(off, slc)], sp_h.at[cid, pl.ds(off, slc)])

    return k(edge, w, degp, xp, zeros_np)


def _tc_tail(sp3, x2, dv2, gates, hid):
    r = x2.shape[0]

    def k(sp_r, x_r, dv_r, g_r, o_r):
        dv = dv_r[...]
        s = dv * (sp_r[0] + sp_r[1]) + dv * dv * x_r[...]

        def body(kk, acc):
            z = jax.nn.sigmoid(s * g_r[0, kk] + g_r[1, kk])
            t = jnp.tanh(s * g_r[2, kk] + g_r[3, kk])
            return acc + jax.nn.relu((1.0 - z) * t) * g_r[4, kk]

        acc = lax.fori_loop(0, hid, body, jnp.zeros_like(s))
        o_r[...] = acc + g_r[5, 0]

    return pl.pallas_call(
        k,
        in_specs=[
            pl.BlockSpec(memory_space=pltpu.VMEM),
            pl.BlockSpec(memory_space=pltpu.VMEM),
            pl.BlockSpec(memory_space=pltpu.VMEM),
            pl.BlockSpec(memory_space=pltpu.SMEM),
        ],
        out_specs=pl.BlockSpec(memory_space=pltpu.VMEM),
        out_shape=jax.ShapeDtypeStruct((r, LANE), jnp.float32),
    )(sp3, x2, dv2, gates)


def kernel(x, edge_index, edge_weight, Wz, bz, Lz_W, Lz_b, Wr, br, Lr_W, Lr_b,
           Wh, bh, Lh_W, Lh_b, Wl, bl):
    n = x.shape[0]
    e = edge_weight.shape[0]
    hid = Wz.shape[1]
    nw = NC * NS

    edge = edge_index.astype(jnp.int32)
    w = edge_weight.astype(jnp.float32)

    ept = -(-e // (nw * 8)) * 8
    if ept * nw != e:
        pad = ept * nw - e
        edge = jnp.pad(edge, ((0, 0), (0, pad)))
        w = jnp.pad(w, (0, pad))
    eoff = ept * nw
    edge = edge.reshape(2 * eoff)

    np_ = -(-n // (NS * LANE)) * (NS * LANE)
    xp = jnp.concatenate([x[:, 0], jnp.zeros((np_ - n,), jnp.float32)])
    zeros_np = jnp.zeros((np_,), jnp.float32)

    degp = _sc_degree(edge, w, zeros_np, np_, ept, eoff)
    sp, dinv = _sc_weighted_agg(edge, w, degp, xp, zeros_np, np_, ept, eoff)

    lzt = Lz_W[:hid]
    lht = Lh_W[:hid]
    az = Wz[0] @ lzt
    cz = bz @ lzt + Lz_b
    ah = Wh[0] @ lht
    ch = bh @ lht + Lh_b
    gates = jnp.stack([az, cz, ah, ch, Wl[:, 0], jnp.full((hid,), bl[0])])

    nr = np_ // LANE
    pred2 = _tc_tail(sp.reshape(NC, nr, LANE), xp.reshape(nr, LANE),
                     dinv.reshape(nr, LANE), gates, hid)
    return pred2.reshape(np_)[:n].reshape(n, 1)


# speedup vs baseline: 1.0027x; 1.0027x over previous
import functools

import jax
import jax.numpy as jnp
from jax import lax
from jax.experimental import pallas as pl
from jax.experimental.pallas import tpu as pltpu
from jax.experimental.pallas import tpu_sc as plsc

NC = 2
NS = 16
LANE = 128
NCH = 4


def _chunks(ept):
    lb = (ept // NCH) // 16 * 16
    lens = [lb] * (NCH - 1) + [ept - (NCH - 1) * lb]
    offs = [i * lb for i in range(NCH)]
    return lens, offs


def _sc_degree(edge, w, zeros_np, np_, ept, eoff):
    slc = np_ // NS
    lens, offs = _chunks(ept)
    mesh = plsc.VectorSubcoreMesh(core_axis_name="c", subcore_axis_name="s",
                                  num_cores=NC, num_subcores=NS)

    @functools.partial(
        pl.kernel,
        out_type=jax.ShapeDtypeStruct((NC, np_), jnp.float32),
        mesh=mesh,
        scratch_types=[
            [pltpu.VMEM((l,), jnp.int32) for l in lens],
            pltpu.VMEM((ept,), jnp.float32),
            pltpu.SemaphoreType.DMA,
            pltpu.SemaphoreType.DMA,
            pltpu.VMEM_SHARED((np_,), jnp.float32),
        ],
    )
    def k(edge_h, w_h, z_h, out_h, col_c, w_v, stsem, ssem, acc):
        cid = lax.axis_index("c")
        sid = lax.axis_index("s")
        base = (sid * NC + cid) * ept
        ds = [pltpu.async_copy(edge_h.at[pl.ds(eoff + base + offs[i], lens[i])],
                               col_c[i], stsem) for i in range(NCH)]
        dw = pltpu.async_copy(w_h.at[pl.ds(base, ept)], w_v, stsem)
        pltpu.sync_copy(z_h.at[pl.ds(sid * slc, slc)], acc.at[pl.ds(sid * slc, slc)])
        for d in ds:
            d.wait()
        dw.wait()
        plsc.subcore_barrier()
        for i in range(NCH):
            pltpu.async_copy(w_v.at[pl.ds(offs[i], lens[i])], acc.at[col_c[i]],
                             ssem, add=True)
        for i in range(NCH):
            pltpu.make_async_copy(w_v.at[pl.ds(offs[i], lens[i])],
                                  acc.at[col_c[i]], ssem).wait()
        plsc.subcore_barrier()
        pltpu.sync_copy(acc.at[pl.ds(sid * slc, slc)],
                        out_h.at[cid, pl.ds(sid * slc, slc)])

    return k(edge, w, zeros_np)


def _sc_weighted_agg(edge, w, degp, xp, zeros_np, np_, ept, eoff):
    slc = np_ // NS
    lens, offs = _chunks(ept)
    lmax = -(-max(lens) // 16) * 16
    mesh = plsc.VectorSubcoreMesh(core_axis_name="c", subcore_axis_name="s",
                                  num_cores=NC, num_subcores=NS)

    @functools.partial(
        pl.kernel,
        out_type=(jax.ShapeDtypeStruct((NC, np_), jnp.float32),
                  jax.ShapeDtypeStruct((np_,), jnp.float32)),
        mesh=mesh,
        scratch_types=[
            [pltpu.VMEM((l,), jnp.int32) for l in lens],
            pltpu.VMEM((ept,), jnp.int32),
            pltpu.VMEM((ept,), jnp.float32),
            pltpu.VMEM((lmax,), jnp.float32),
            pltpu.VMEM((lmax,), jnp.float32),
            pltpu.VMEM((lmax,), jnp.float32),
            pltpu.VMEM((lmax,), jnp.float32),
            pltpu.VMEM((slc,), jnp.float32),
            pltpu.VMEM((slc,), jnp.float32),
            pltpu.VMEM((slc,), jnp.float32),
            pltpu.VMEM((slc,), jnp.float32),
            pltpu.VMEM((slc,), jnp.float32),
            pltpu.SemaphoreType.DMA,
            pltpu.SemaphoreType.DMA,
            pltpu.SemaphoreType.DMA,
            pltpu.SemaphoreType.DMA,
            pltpu.SemaphoreType.DMA,
            pltpu.VMEM_SHARED((np_,), jnp.float32),
            pltpu.VMEM_SHARED((np_,), jnp.float32),
        ],
    )
    def k(edge_h, w_h, degp_h, x_h, z_h, sp_h, dinv_h,
          col_c, row_v, w_v, uvA, uvB, tvA, tvB, p0v, p1v, xcv, ucv, dcv,
          stsem, gA, gB, sA, sB, u_sh, acc):
        cid = lax.axis_index("c")
        sid = lax.axis_index("s")
        base = (sid * NC + cid) * ept
        off = sid * slc
        dstage = [pltpu.async_copy(edge_h.at[pl.ds(eoff + base + offs[i], lens[i])],
                                   col_c[i], stsem) for i in range(NCH)]
        dstage.append(pltpu.async_copy(edge_h.at[pl.ds(base, ept)], row_v,
                                       stsem))
        dstage.append(pltpu.async_copy(w_h.at[pl.ds(base, ept)], w_v, stsem))
        pltpu.sync_copy(z_h.at[pl.ds(off, slc)], acc.at[pl.ds(off, slc)])
        pltpu.sync_copy(degp_h.at[0, pl.ds(off, slc)], p0v)
        pltpu.sync_copy(degp_h.at[1, pl.ds(off, slc)], p1v)
        pltpu.sync_copy(x_h.at[pl.ds(off, slc)], xcv)

        def newton(i, c):
            sl = pl.ds(i * 16, 16)
            d = p0v[sl] + p1v[sl] + 1.0
            iv = lax.bitcast_convert_type(d, jnp.int32)
            y = lax.bitcast_convert_type(
                jnp.full((16,), 0x5F3759DF, jnp.int32) - jnp.right_shift(iv, 1),
                jnp.float32)
            for _ in range(3):
                y = y * (1.5 - 0.5 * d * y * y)
            dcv[sl] = y
            ucv[sl] = y * xcv[sl]
            return c

        lax.fori_loop(0, slc // 16, newton, 0)
        pltpu.sync_copy(ucv, u_sh.at[pl.ds(off, slc)])

        @pl.when(cid == 0)
        def _():
            pltpu.sync_copy(dcv, dinv_h.at[pl.ds(off, slc)])

        plsc.subcore_barrier()
        for d in dstage:
            d.wait()

        def g_fire(ci, uvX, sem):
            return pltpu.async_copy(
                u_sh.at[row_v.at[pl.ds(offs[ci], lens[ci])]],
                uvX.at[pl.ds(0, lens[ci])], sem)

        def s_fire(ci, tvX, sem):
            return pltpu.async_copy(tvX.at[pl.ds(0, lens[ci])],
                                    acc.at[col_c[ci]], sem, add=True)

        def s_drain(ci, tvX, sem):
            pltpu.make_async_copy(tvX.at[pl.ds(0, lens[ci])],
                                  acc.at[col_c[ci]], sem).wait()

        def mul(ci, uvX, tvX):
            n16 = -(-lens[ci] // 16)

            def b(i, c):
                sl = pl.ds(i * 16, 16)
                tvX[sl] = w_v[pl.ds(offs[ci] + i * 16, 16)] * uvX[sl]
                return c

            lax.fori_loop(0, n16, b, 0)

        ga = g_fire(0, uvA, gA)
        gb = g_fire(1, uvB, gB)
        for c in range(0, NCH, 2):
            ga.wait()
            if c >= 2:
                s_drain(c - 2, tvA, sA)
            mul(c, uvA, tvA)
            s_fire(c, tvA, sA)
            if c + 2 < NCH:
                ga = g_fire(c + 2, uvA, gA)
            gb.wait()
            if c >= 2:
                s_drain(c - 1, tvB, sB)
            mul(c + 1, uvB, tvB)
            s_fire(c + 1, tvB, sB)
            if c + 3 < NCH:
                gb = g_fire(c + 3, uvB, gB)
        s_drain(NCH - 2, tvA, sA)
        s_drain(NCH - 1, tvB, sB)
        plsc.subcore_barrier()
        pltpu.sync_copy(acc.at[pl.ds(off, slc)], sp_h.at[cid, pl.ds(off, slc)])

    return k(edge, w, degp, xp, zeros_np)


def _tc_tail(sp3, x2, dv2, gates, hid):
    r = x2.shape[0]

    def k(sp_r, x_r, dv_r, g_r, o_r):
        dv = dv_r[...]
        s = dv * (sp_r[0] + sp_r[1]) + dv * dv * x_r[...]

        def body(kk, acc):
            z = jax.nn.sigmoid(s * g_r[0, kk] + g_r[1, kk])
            t = jnp.tanh(s * g_r[2, kk] + g_r[3, kk])
            return acc + jax.nn.relu((1.0 - z) * t) * g_r[4, kk]

        acc = lax.fori_loop(0, hid, body, jnp.zeros_like(s))
        o_r[...] = acc + g_r[5, 0]

    return pl.pallas_call(
        k,
        in_specs=[
            pl.BlockSpec(memory_space=pltpu.VMEM),
            pl.BlockSpec(memory_space=pltpu.VMEM),
            pl.BlockSpec(memory_space=pltpu.VMEM),
            pl.BlockSpec(memory_space=pltpu.SMEM),
        ],
        out_specs=pl.BlockSpec(memory_space=pltpu.VMEM),
        out_shape=jax.ShapeDtypeStruct((r, LANE), jnp.float32),
    )(sp3, x2, dv2, gates)


def kernel(x, edge_index, edge_weight, Wz, bz, Lz_W, Lz_b, Wr, br, Lr_W, Lr_b,
           Wh, bh, Lh_W, Lh_b, Wl, bl):
    n = x.shape[0]
    e = edge_weight.shape[0]
    hid = Wz.shape[1]
    nw = NC * NS

    edge = edge_index.astype(jnp.int32)
    w = edge_weight.astype(jnp.float32)

    ept = -(-e // (nw * 8)) * 8
    if ept * nw != e:
        pad = ept * nw - e
        edge = jnp.pad(edge, ((0, 0), (0, pad)))
        w = jnp.pad(w, (0, pad))
    eoff = ept * nw
    edge = edge.reshape(2 * eoff)

    np_ = -(-n // (NS * LANE)) * (NS * LANE)
    xp = jnp.concatenate([x[:, 0], jnp.zeros((np_ - n,), jnp.float32)])
    zeros_np = jnp.zeros((np_,), jnp.float32)

    degp = _sc_degree(edge, w, zeros_np, np_, ept, eoff)
    sp, dinv = _sc_weighted_agg(edge, w, degp, xp, zeros_np, np_, ept, eoff)

    lzt = Lz_W[:hid]
    lht = Lh_W[:hid]
    az = Wz[0] @ lzt
    cz = bz @ lzt + Lz_b
    ah = Wh[0] @ lht
    ch = bh @ lht + Lh_b
    gates = jnp.stack([az, cz, ah, ch, Wl[:, 0], jnp.full((hid,), bl[0])])

    nr = np_ // LANE
    pred2 = _tc_tail(sp.reshape(NC, nr, LANE), xp.reshape(nr, LANE),
                     dinv.reshape(nr, LANE), gates, hid)
    return pred2.reshape(np_)[:n].reshape(n, 1)
